# Initial kernel scaffold; baseline (speedup 1.0000x reference)
import jax
import jax.numpy as jnp
from jax.experimental import pallas as pl


def kernel(input, conn_in, conn_state, conn_out, mem_in, mem_state, mem_out):
    raise NotImplementedError("write your pallas kernel here")


# speedup vs baseline: 2.1803x; 2.1803x over previous
import functools

import jax
import jax.numpy as jnp
from jax import lax
from jax.experimental import pallas as pl
from jax.experimental.pallas import tpu as pltpu
from jax.experimental.pallas import tpu_sc as plsc

_B = 1024
_NB = 12
_L = 16
_NW = 32
_G = 4


def _ram_layer_sc(bitsT, conn, mem):
    N = conn.shape[0]
    conn_flat = conn.reshape(N * _NB)
    npw = N // _NW
    nch = npw // _G
    mesh = plsc.VectorSubcoreMesh(core_axis_name="c", subcore_axis_name="s")

    @functools.partial(
        pl.kernel,
        out_type=jax.ShapeDtypeStruct((N, _B), jnp.int32),
        mesh=mesh,
        scratch_types=[
            pltpu.VMEM((npw * _NB,), jnp.int32),
            pltpu.VMEM((_G * _NB, _B), jnp.int32),
            pltpu.VMEM((_G, 4096), jnp.int32),
            pltpu.VMEM((_G, _B), jnp.int32),
            pltpu.SemaphoreType.DMA,
        ],
        compiler_params=pltpu.CompilerParams(needs_layout_passes=False),
    )
    def layer(bitsT_hbm, conn_hbm, mem_hbm, out_hbm,
              conn_v, cols_v, mem_v, out_v, sem):
        wid = lax.axis_index("s") * 2 + lax.axis_index("c")
        base = wid * npw
        pltpu.sync_copy(conn_hbm.at[pl.ds(base * _NB, npw * _NB)], conn_v)

        def chunk(c, carry):
            n0 = base + c * _G
            idx = conn_v.at[pl.ds(c * (_G * _NB), _G * _NB)]
            cp = pltpu.async_copy(bitsT_hbm.at[idx], cols_v, sem)
            pltpu.sync_copy(mem_hbm.at[pl.ds(n0, _G)], mem_v)
            cp.wait()

            def group(t, carry2):
                sl = pl.ds(t * _L, _L)
                for j in range(_G):
                    addr = cols_v[j * _NB, sl]
                    for k in range(1, _NB):
                        addr = addr | (cols_v[j * _NB + k, sl] << k)
                    row = jnp.full((_L,), j, jnp.int32)
                    bit = plsc.load_gather(mem_v, [row, addr & 4095])
                    out_v[j, sl] = bit
                return carry2

            lax.fori_loop(0, _B // _L, group, 0)
            pltpu.sync_copy(out_v, out_hbm.at[pl.ds(n0, _G)])
            return carry

        lax.fori_loop(0, nch, chunk, 0)

    return layer(bitsT, conn_flat, mem)


def kernel(input, conn_in, conn_state, conn_out, mem_in, mem_state, mem_out):
    bitsT = input.T.astype(jnp.int32)
    out1T = _ram_layer_sc(bitsT, conn_in, mem_in.astype(jnp.int32))
    zerosT = jnp.zeros_like(out1T)
    bitsT2 = jnp.concatenate([out1T, zerosT], axis=0)
    out2T = _ram_layer_sc(bitsT2, conn_state, mem_state.astype(jnp.int32))
    bitsT3 = jnp.concatenate([out1T, out2T], axis=0)
    outT = _ram_layer_sc(bitsT3, conn_out, mem_out.astype(jnp.int32))
    return outT.T.astype(jnp.bool_)
